# Initial kernel scaffold; baseline (speedup 1.0000x reference)
import jax
import jax.numpy as jnp
from jax.experimental import pallas as pl


def kernel(events, embed, W1, b1, W2, b2):
    raise NotImplementedError("write your pallas kernel here")


# speedup vs baseline: 22.1682x; 22.1682x over previous
import functools

import numpy as np
import jax
import jax.numpy as jnp
from jax.experimental import pallas as pl
from jax.experimental.pallas import tpu as pltpu
from jax.experimental.pallas import tpu_sc as plsc

D_MODEL = 16
N_FIELDS = 26
TOTAL = D_MODEL * N_FIELDS

_GATHER_WINDOW = 128
_ROW_BLK = 1280


def _gather_rows(embed, idx_flat, n_idx):
    mesh = plsc.VectorSubcoreMesh(core_axis_name="core", subcore_axis_name="subcore")

    @functools.partial(
        pl.kernel,
        out_type=jax.ShapeDtypeStruct((n_idx, D_MODEL), embed.dtype),
        mesh=mesh,
        compiler_params=pltpu.CompilerParams(use_tc_tiling_on_sc=False),
    )
    def gather_kernel(table_hbm, i_hbm, o_hbm):
        def body(i_vmem, o_vmem):
            pltpu.sync_copy(table_hbm.at[i_vmem.at[0]], o_vmem)

        pltpu.emit_pipeline(
            body,
            grid=(n_idx // _GATHER_WINDOW,),
            in_specs=[
                pl.BlockSpec((1, _GATHER_WINDOW), index_map=lambda i: (0, i))
            ],
            out_specs=[
                pl.BlockSpec((_GATHER_WINDOW, D_MODEL), index_map=lambda i: (i, 0))
            ],
            core_axis_name=("core", "subcore"),
            dimension_semantics=(pltpu.PARALLEL,),
        )(i_hbm, o_hbm)

    return gather_kernel(embed, idx_flat)


def _mlp_body(x_ref, w1_ref, b1_ref, w2_ref, peb2_ref, o_ref):
    x = x_ref[...].astype(jnp.bfloat16)
    h = jnp.dot(x, w1_ref[...], preferred_element_type=jnp.float32)
    h = h + b1_ref[...]
    t = jnp.exp(jnp.minimum(h, 20.0))
    u = t * (t + 2.0)
    m = h * (u / (u + 2.0))
    o_ref[...] = (
        jnp.dot(m.astype(jnp.bfloat16), w2_ref[...], preferred_element_type=jnp.float32)
        + peb2_ref[...]
    )


def _pe_tile(hist, rows):
    pos = np.arange(hist, dtype=np.float32)[:, None]
    div = np.exp(
        np.arange(0, D_MODEL, 2, dtype=np.float32) * (-np.log(10000.0) / D_MODEL)
    )
    pe = np.zeros((hist, D_MODEL), dtype=np.float32)
    pe[:, 0::2] = np.sin(pos * div)
    pe[:, 1::2] = np.cos(pos * div)
    return np.tile(pe, (rows // hist, 1))


def kernel(events, embed, W1, b1, W2, b2):
    batch, hist, n_fields = events.shape
    n_rows = batch * hist
    n_idx = n_rows * n_fields

    idx_flat = events.reshape(1, n_idx)
    gathered = _gather_rows(embed, idx_flat, n_idx)
    x = gathered.reshape(n_rows, TOTAL)

    w1b = W1.astype(jnp.bfloat16)
    w2b = W2.astype(jnp.bfloat16)
    b1r = b1.reshape(1, TOTAL * 4)
    peb2 = jnp.asarray(_pe_tile(hist, _ROW_BLK)) + b2[None, :]

    out = pl.pallas_call(
        _mlp_body,
        grid=(n_rows // _ROW_BLK,),
        in_specs=[
            pl.BlockSpec((_ROW_BLK, TOTAL), lambda i: (i, 0)),
            pl.BlockSpec((TOTAL, TOTAL * 4), lambda i: (0, 0)),
            pl.BlockSpec((1, TOTAL * 4), lambda i: (0, 0)),
            pl.BlockSpec((TOTAL * 4, D_MODEL), lambda i: (0, 0)),
            pl.BlockSpec((_ROW_BLK, D_MODEL), lambda i: (0, 0)),
        ],
        out_specs=pl.BlockSpec((_ROW_BLK, D_MODEL), lambda i: (i, 0)),
        out_shape=jax.ShapeDtypeStruct((n_rows, D_MODEL), jnp.float32),
    )(x, w1b, b1r, w2b, peb2)

    return out.reshape(batch, hist, D_MODEL)


# speedup vs baseline: 22.1738x; 1.0003x over previous
import functools

import numpy as np
import jax
import jax.numpy as jnp
from jax.experimental import pallas as pl
from jax.experimental.pallas import tpu as pltpu
from jax.experimental.pallas import tpu_sc as plsc

D_MODEL = 16
N_FIELDS = 26
TOTAL = D_MODEL * N_FIELDS

_GATHER_WINDOW = 128
_ROW_BLK = 1280


def _gather_rows(embed, idx_flat, n_idx):
    mesh = plsc.VectorSubcoreMesh(core_axis_name="core", subcore_axis_name="subcore")

    @functools.partial(
        pl.kernel,
        out_type=jax.ShapeDtypeStruct((n_idx, D_MODEL), embed.dtype),
        mesh=mesh,
        compiler_params=pltpu.CompilerParams(use_tc_tiling_on_sc=False),
    )
    def gather_kernel(table_hbm, i_hbm, o_hbm):
        def body(i_vmem, o_vmem):
            pltpu.sync_copy(table_hbm.at[i_vmem], o_vmem)

        pltpu.emit_pipeline(
            body,
            grid=(n_idx // _GATHER_WINDOW,),
            in_specs=[
                pl.BlockSpec((_GATHER_WINDOW,), index_map=lambda i: (i,))
            ],
            out_specs=[
                pl.BlockSpec((_GATHER_WINDOW, D_MODEL), index_map=lambda i: (i, 0))
            ],
            core_axis_name=("core", "subcore"),
            dimension_semantics=(pltpu.PARALLEL,),
        )(i_hbm, o_hbm)

    return gather_kernel(embed, idx_flat)


def _mlp_body(x_ref, w1_ref, b1_ref, w2_ref, peb2_ref, o_ref):
    x = x_ref[...].astype(jnp.bfloat16)
    h = jnp.dot(x, w1_ref[...], preferred_element_type=jnp.float32)
    h = h + b1_ref[...]
    t = jnp.exp(jnp.minimum(h, 20.0))
    u = t * (t + 2.0)
    m = h * (u / (u + 2.0))
    o_ref[...] = (
        jnp.dot(m.astype(jnp.bfloat16), w2_ref[...], preferred_element_type=jnp.float32)
        + peb2_ref[...]
    )


def _pe_tile(hist, rows):
    pos = np.arange(hist, dtype=np.float32)[:, None]
    div = np.exp(
        np.arange(0, D_MODEL, 2, dtype=np.float32) * (-np.log(10000.0) / D_MODEL)
    )
    pe = np.zeros((hist, D_MODEL), dtype=np.float32)
    pe[:, 0::2] = np.sin(pos * div)
    pe[:, 1::2] = np.cos(pos * div)
    return np.tile(pe, (rows // hist, 1))


def kernel(events, embed, W1, b1, W2, b2):
    batch, hist, n_fields = events.shape
    n_rows = batch * hist
    n_idx = n_rows * n_fields

    idx_flat = events.reshape(n_idx)
    gathered = _gather_rows(embed, idx_flat, n_idx)
    x = gathered.reshape(n_rows, TOTAL)

    w1b = W1.astype(jnp.bfloat16)
    w2b = W2.astype(jnp.bfloat16)
    b1r = b1.reshape(1, TOTAL * 4)
    peb2 = jnp.asarray(_pe_tile(hist, _ROW_BLK)) + b2[None, :]

    out = pl.pallas_call(
        _mlp_body,
        grid=(n_rows // _ROW_BLK,),
        in_specs=[
            pl.BlockSpec((_ROW_BLK, TOTAL), lambda i: (i, 0)),
            pl.BlockSpec((TOTAL, TOTAL * 4), lambda i: (0, 0)),
            pl.BlockSpec((1, TOTAL * 4), lambda i: (0, 0)),
            pl.BlockSpec((TOTAL * 4, D_MODEL), lambda i: (0, 0)),
            pl.BlockSpec((_ROW_BLK, D_MODEL), lambda i: (0, 0)),
        ],
        out_specs=pl.BlockSpec((_ROW_BLK, D_MODEL), lambda i: (i, 0)),
        out_shape=jax.ShapeDtypeStruct((n_rows, D_MODEL), jnp.float32),
    )(x, w1b, b1r, w2b, peb2)

    return out.reshape(batch, hist, D_MODEL)


# speedup vs baseline: 26.2899x; 1.1856x over previous
import functools

import numpy as np
import jax
import jax.numpy as jnp
from jax.experimental import pallas as pl
from jax.experimental.pallas import tpu as pltpu
from jax.experimental.pallas import tpu_sc as plsc

D_MODEL = 16
N_FIELDS = 26
TOTAL = D_MODEL * N_FIELDS

_GATHER_WINDOW = 512
_ROW_BLK = 1280


def _gather_rows(embed, idx_flat, n_idx):
    mesh = plsc.VectorSubcoreMesh(core_axis_name="core", subcore_axis_name="subcore")

    @functools.partial(
        pl.kernel,
        out_type=jax.ShapeDtypeStruct((n_idx, D_MODEL), embed.dtype),
        mesh=mesh,
        compiler_params=pltpu.CompilerParams(use_tc_tiling_on_sc=False),
    )
    def gather_kernel(table_hbm, i_hbm, o_hbm):
        def body(i_vmem, o_vmem):
            pltpu.sync_copy(table_hbm.at[i_vmem], o_vmem)

        pltpu.emit_pipeline(
            body,
            grid=(n_idx // _GATHER_WINDOW,),
            in_specs=[
                pl.BlockSpec((_GATHER_WINDOW,), index_map=lambda i: (i,))
            ],
            out_specs=[
                pl.BlockSpec((_GATHER_WINDOW, D_MODEL), index_map=lambda i: (i, 0))
            ],
            core_axis_name=("core", "subcore"),
            dimension_semantics=(pltpu.PARALLEL,),
        )(i_hbm, o_hbm)

    return gather_kernel(embed, idx_flat)


def _mlp_body(x_ref, w1_ref, b1_ref, w2_ref, peb2_ref, o_ref):
    x = x_ref[...].astype(jnp.bfloat16)
    h = jnp.dot(x, w1_ref[...], preferred_element_type=jnp.float32)
    h = h + b1_ref[...]
    t = jnp.exp(jnp.minimum(h, 20.0))
    u = t * (t + 2.0)
    m = h * (u / (u + 2.0))
    o_ref[...] = (
        jnp.dot(m.astype(jnp.bfloat16), w2_ref[...], preferred_element_type=jnp.float32)
        + peb2_ref[...]
    )


def _pe_tile(hist, rows):
    pos = np.arange(hist, dtype=np.float32)[:, None]
    div = np.exp(
        np.arange(0, D_MODEL, 2, dtype=np.float32) * (-np.log(10000.0) / D_MODEL)
    )
    pe = np.zeros((hist, D_MODEL), dtype=np.float32)
    pe[:, 0::2] = np.sin(pos * div)
    pe[:, 1::2] = np.cos(pos * div)
    return np.tile(pe, (rows // hist, 1))


def kernel(events, embed, W1, b1, W2, b2):
    batch, hist, n_fields = events.shape
    n_rows = batch * hist
    n_idx = n_rows * n_fields

    idx_flat = events.reshape(n_idx)
    gathered = _gather_rows(embed, idx_flat, n_idx)
    x = gathered.reshape(n_rows, TOTAL)

    w1b = W1.astype(jnp.bfloat16)
    w2b = W2.astype(jnp.bfloat16)
    b1r = b1.reshape(1, TOTAL * 4)
    peb2 = jnp.asarray(_pe_tile(hist, _ROW_BLK)) + b2[None, :]

    out = pl.pallas_call(
        _mlp_body,
        grid=(n_rows // _ROW_BLK,),
        in_specs=[
            pl.BlockSpec((_ROW_BLK, TOTAL), lambda i: (i, 0)),
            pl.BlockSpec((TOTAL, TOTAL * 4), lambda i: (0, 0)),
            pl.BlockSpec((1, TOTAL * 4), lambda i: (0, 0)),
            pl.BlockSpec((TOTAL * 4, D_MODEL), lambda i: (0, 0)),
            pl.BlockSpec((_ROW_BLK, D_MODEL), lambda i: (0, 0)),
        ],
        out_specs=pl.BlockSpec((_ROW_BLK, D_MODEL), lambda i: (i, 0)),
        out_shape=jax.ShapeDtypeStruct((n_rows, D_MODEL), jnp.float32),
    )(x, w1b, b1r, w2b, peb2)

    return out.reshape(batch, hist, D_MODEL)


# speedup vs baseline: 27.8026x; 1.0575x over previous
import functools

import numpy as np
import jax
import jax.numpy as jnp
from jax.experimental import pallas as pl
from jax.experimental.pallas import tpu as pltpu
from jax.experimental.pallas import tpu_sc as plsc

D_MODEL = 16
N_FIELDS = 26
TOTAL = D_MODEL * N_FIELDS

_GATHER_WINDOW = 2048
_ROW_BLK = 1280


def _gather_rows(embed, idx_flat, n_idx):
    mesh = plsc.VectorSubcoreMesh(core_axis_name="core", subcore_axis_name="subcore")

    @functools.partial(
        pl.kernel,
        out_type=jax.ShapeDtypeStruct((n_idx, D_MODEL), embed.dtype),
        mesh=mesh,
        compiler_params=pltpu.CompilerParams(use_tc_tiling_on_sc=False),
    )
    def gather_kernel(table_hbm, i_hbm, o_hbm):
        def body(i_vmem, o_vmem):
            pltpu.sync_copy(table_hbm.at[i_vmem], o_vmem)

        pltpu.emit_pipeline(
            body,
            grid=(n_idx // _GATHER_WINDOW,),
            in_specs=[
                pl.BlockSpec((_GATHER_WINDOW,), index_map=lambda i: (i,))
            ],
            out_specs=[
                pl.BlockSpec((_GATHER_WINDOW, D_MODEL), index_map=lambda i: (i, 0))
            ],
            core_axis_name=("core", "subcore"),
            dimension_semantics=(pltpu.PARALLEL,),
        )(i_hbm, o_hbm)

    return gather_kernel(embed, idx_flat)


def _mlp_body(x_ref, w1_ref, b1_ref, w2_ref, peb2_ref, o_ref):
    x = x_ref[...].astype(jnp.bfloat16)
    h = jnp.dot(x, w1_ref[...], preferred_element_type=jnp.float32)
    h = h + b1_ref[...]
    t = jnp.exp(jnp.minimum(h, 20.0))
    u = t * (t + 2.0)
    m = h * (u / (u + 2.0))
    o_ref[...] = (
        jnp.dot(m.astype(jnp.bfloat16), w2_ref[...], preferred_element_type=jnp.float32)
        + peb2_ref[...]
    )


def _pe_tile(hist, rows):
    pos = np.arange(hist, dtype=np.float32)[:, None]
    div = np.exp(
        np.arange(0, D_MODEL, 2, dtype=np.float32) * (-np.log(10000.0) / D_MODEL)
    )
    pe = np.zeros((hist, D_MODEL), dtype=np.float32)
    pe[:, 0::2] = np.sin(pos * div)
    pe[:, 1::2] = np.cos(pos * div)
    return np.tile(pe, (rows // hist, 1))


def kernel(events, embed, W1, b1, W2, b2):
    batch, hist, n_fields = events.shape
    n_rows = batch * hist
    n_idx = n_rows * n_fields

    idx_flat = events.reshape(n_idx)
    gathered = _gather_rows(embed, idx_flat, n_idx)
    x = gathered.reshape(n_rows, TOTAL)

    w1b = W1.astype(jnp.bfloat16)
    w2b = W2.astype(jnp.bfloat16)
    b1r = b1.reshape(1, TOTAL * 4)
    peb2 = jnp.asarray(_pe_tile(hist, _ROW_BLK)) + b2[None, :]

    out = pl.pallas_call(
        _mlp_body,
        grid=(n_rows // _ROW_BLK,),
        in_specs=[
            pl.BlockSpec((_ROW_BLK, TOTAL), lambda i: (i, 0)),
            pl.BlockSpec((TOTAL, TOTAL * 4), lambda i: (0, 0)),
            pl.BlockSpec((1, TOTAL * 4), lambda i: (0, 0)),
            pl.BlockSpec((TOTAL * 4, D_MODEL), lambda i: (0, 0)),
            pl.BlockSpec((_ROW_BLK, D_MODEL), lambda i: (0, 0)),
        ],
        out_specs=pl.BlockSpec((_ROW_BLK, D_MODEL), lambda i: (i, 0)),
        out_shape=jax.ShapeDtypeStruct((n_rows, D_MODEL), jnp.float32),
    )(x, w1b, b1r, w2b, peb2)

    return out.reshape(batch, hist, D_MODEL)
